# baseline (speedup 1.0000x reference)
import functools

import jax
import jax.numpy as jnp
from jax.experimental import pallas as pl
from jax.experimental.pallas import tpu as pltpu

NU = 50000
NM = 50000
E = 800000
EL = 200000
DF = 128
H = 64


def _sage_x(x_src, x_dst, src, dst, Wl, bl, Wr, n_dst):
    msg = jnp.take(x_src, src, axis=0)
    agg = jax.ops.segment_sum(msg, dst, num_segments=n_dst)
    cnt = jax.ops.segment_sum(jnp.ones((src.shape[0],), x_src.dtype), dst,
                              num_segments=n_dst)
    mean = agg / jnp.maximum(cnt, 1.0)[:, None]
    return mean @ Wl + bl + x_dst @ Wr


def _normalize(x):
    n = jnp.sqrt(jnp.sum(x * x, axis=-1, keepdims=True))
    return x / jnp.maximum(n, 1e-12)


def _dec_kernel(u_ref, m_ref, w1_ref, b1_ref, w2_ref, b2_ref, o_ref):
    x = jnp.concatenate([u_ref[...], m_ref[...]], axis=-1)
    h = jax.nn.relu(jnp.dot(x, w1_ref[...],
                            preferred_element_type=jnp.float32) + b1_ref[...])
    o_ref[...] = jnp.dot(h, w2_ref[...],
                         preferred_element_type=jnp.float32) + b2_ref[0, 0]


def _decoder(u, m, W1, b1, W2, b2):
    BK = 2000
    grid = (EL // BK,)
    return pl.pallas_call(
        _dec_kernel,
        grid=grid,
        in_specs=[
            pl.BlockSpec((BK, H), lambda i: (i, 0)),
            pl.BlockSpec((BK, H), lambda i: (i, 0)),
            pl.BlockSpec((2 * H, H), lambda i: (0, 0)),
            pl.BlockSpec((1, H), lambda i: (0, 0)),
            pl.BlockSpec((H, 1), lambda i: (0, 0)),
            pl.BlockSpec((1, 1), lambda i: (0, 0)),
        ],
        out_specs=pl.BlockSpec((BK, 1), lambda i: (i, 0)),
        out_shape=jax.ShapeDtypeStruct((EL, 1), jnp.float32),
    )(u, m, W1, b1.reshape(1, H), W2, b2.reshape(1, 1))


def kernel(movie_x, edge_index, edge_label_index, user_emb_w, proj_W, proj_b,
           c1_um_Wl, c1_um_bl, c1_um_Wr, c1_mu_Wl, c1_mu_bl, c1_mu_Wr,
           c2_um_Wl, c2_um_bl, c2_um_Wr, c2_mu_Wl, c2_mu_bl, c2_mu_Wr,
           dec_W1, dec_b1, dec_W2, dec_b2):
    x_user = user_emb_w
    x_movie = jax.nn.relu(movie_x @ proj_W + proj_b)
    src = edge_index[0]
    dst = edge_index[1]
    m1 = jax.nn.relu(_sage_x(x_user, x_movie, src, dst, c1_um_Wl, c1_um_bl,
                             c1_um_Wr, NM))
    u1 = jax.nn.relu(_sage_x(x_movie, x_user, dst, src, c1_mu_Wl, c1_mu_bl,
                             c1_mu_Wr, NU))
    ru = x_user + u1
    rm = x_movie + m1
    m2 = _sage_x(ru, rm, src, dst, c2_um_Wl, c2_um_bl, c2_um_Wr, NM)
    u2 = _sage_x(rm, ru, dst, src, c2_mu_Wl, c2_mu_bl, c2_mu_Wr, NU)
    un = _normalize(u2)
    mn = _normalize(m2)
    u = jnp.take(un, edge_label_index[0], axis=0)
    m = jnp.take(mn, edge_label_index[1], axis=0)
    return _decoder(u, m, dec_W1, dec_b1, dec_W2, dec_b2)


# speedup vs baseline: 3.8345x; 3.8345x over previous
import functools

import jax
import jax.numpy as jnp
from jax import lax
from jax.experimental import pallas as pl
from jax.experimental.pallas import tpu as pltpu
from jax.experimental.pallas import tpu_sc as plsc

NU = 50000
NM = 50000
E = 800000
EL = 200000
DF = 128
H = 64

NS = 16
CH = 1024
NCH_E = (E + CH - 1) // CH
EPAD = NCH_E * CH
NT_E = (NCH_E + NS - 1) // NS
NCH_L = (EL + CH - 1) // CH
LPAD = NCH_L * CH
NT_L = (NCH_L + NS - 1) // NS
NP = 51200
RPS = NP // NS
JUNK = 50000
NPH = 25600
ACCR = 26112
JUNK2 = 26000
ZRS = ACCR // NS // 2
DPS = NPH // NS

_mesh = plsc.VectorSubcoreMesh(core_axis_name="c", subcore_axis_name="s")
_sc_params = pltpu.CompilerParams(use_tc_tiling_on_sc=False)


def _zero_f32(ref, rows):
    z = jnp.zeros((16,), jnp.float32)

    @pl.loop(0, rows)
    def _(i):
        ref[i, pl.ds(0, 16)] = z
        ref[i, pl.ds(16, 16)] = z


@functools.partial(
    pl.kernel,
    mesh=_mesh,
    compiler_params=_sc_params,
    out_type=jax.ShapeDtypeStruct((2, NP, 32), jnp.float32),
    scratch_types=[
        pltpu.VMEM((CH,), jnp.int32),
        pltpu.VMEM((8, 128), jnp.int32),
        pltpu.VMEM((CH, 32), jnp.float32),
        pltpu.VMEM((ZRS, 32), jnp.float32),
        pltpu.VMEM_SHARED((ACCR, 32), jnp.float32),
        pltpu.SemaphoreType.DMA,
    ],
)
def _sc_segsum(tab, gidx, sidx, out, gv, dv, rows, zbuf, acc, sem):
    c = lax.axis_index("c")
    s = lax.axis_index("s")

    _zero_f32(zbuf, ZRS)
    for p in range(2):
        for q in range(2):
            pltpu.sync_copy(zbuf,
                            acc.at[pl.ds((s * 2 + q) * ZRS, ZRS)])
        plsc.subcore_barrier()

        @pl.loop(0, NT_E)
        def _(t):
            j = t * NS + s

            @pl.when(j < NCH_E)
            def _():
                pltpu.sync_copy(gidx.at[c, pl.ds(j * CH, CH)], gv)
                pltpu.async_copy(tab.at[gv], rows, sem).wait()
                pltpu.sync_copy(sidx.at[p, pl.ds(j * 8, 8)], dv)
                for b in range(8):
                    pltpu.sync_copy(rows.at[pl.ds(b * 128, 128)],
                                    acc.at[dv.at[b]], add=True)

        plsc.subcore_barrier()
        pltpu.sync_copy(acc.at[pl.ds(s * DPS, DPS)],
                        out.at[c, pl.ds(p * NPH + s * DPS, DPS)])
        plsc.subcore_barrier()


@functools.partial(
    pl.kernel,
    mesh=_mesh,
    compiler_params=_sc_params,
    out_type=jax.ShapeDtypeStruct((2, NP, 16), jnp.float32),
    scratch_types=[
        pltpu.VMEM((8, 128), jnp.int32),
        pltpu.VMEM((128, 16), jnp.float32),
        pltpu.VMEM((RPS, 16), jnp.float32),
        pltpu.VMEM_SHARED((NP, 16), jnp.float32),
    ],
)
def _sc_hist(sidx, out, dv, ones, zbuf, acc):
    c = lax.axis_index("c")
    s = lax.axis_index("s")

    one = jnp.ones((16,), jnp.float32)
    zero = jnp.zeros((16,), jnp.float32)

    @pl.loop(0, 128)
    def _(i):
        ones[i, pl.ds(0, 16)] = one

    @pl.loop(0, RPS)
    def _(i):
        zbuf[i, pl.ds(0, 16)] = zero

    pltpu.sync_copy(zbuf, acc.at[pl.ds(s * RPS, RPS)])
    plsc.subcore_barrier()

    @pl.loop(0, NT_E)
    def _(t):
        j = t * NS + s

        @pl.when(j < NCH_E)
        def _():
            pltpu.sync_copy(sidx.at[c, pl.ds(j * 8, 8)], dv)
            for b in range(8):
                pltpu.sync_copy(ones, acc.at[dv.at[b]], add=True)

    plsc.subcore_barrier()
    pltpu.sync_copy(acc.at[pl.ds(s * RPS, RPS)],
                    out.at[c, pl.ds(s * RPS, RPS)])


@functools.partial(
    pl.kernel,
    mesh=_mesh,
    compiler_params=_sc_params,
    out_type=jax.ShapeDtypeStruct((2, LPAD, 32), jnp.float32),
    scratch_types=[
        pltpu.VMEM((CH,), jnp.int32),
        pltpu.VMEM((CH,), jnp.int32),
        pltpu.VMEM((CH, 32), jnp.float32),
        pltpu.VMEM((CH, 32), jnp.float32),
        pltpu.SemaphoreType.DMA,
        pltpu.SemaphoreType.DMA,
    ],
)
def _sc_decgather(pu, pm, i0, i1, out, iv0, iv1, bu, bm, sem0, sem1):
    c = lax.axis_index("c")
    s = lax.axis_index("s")

    @pl.loop(0, NT_L)
    def _(t):
        j = t * NS + s

        @pl.when(j < NCH_L)
        def _():
            pltpu.sync_copy(i0.at[c, pl.ds(j * CH, CH)], iv0)
            cp0 = pltpu.async_copy(pu.at[iv0], bu, sem0)
            pltpu.sync_copy(i1.at[c, pl.ds(j * CH, CH)], iv1)
            cp1 = pltpu.async_copy(pm.at[iv1], bm, sem1)
            cp0.wait()
            cp1.wait()

            @pl.loop(0, CH)
            def _(r):
                bu[r, pl.ds(0, 16)] = bu[r, pl.ds(0, 16)] + bm[r, pl.ds(0, 16)]
                bu[r, pl.ds(16, 16)] = (bu[r, pl.ds(16, 16)]
                                        + bm[r, pl.ds(16, 16)])

            pltpu.sync_copy(bu, out.at[c, pl.ds(j * CH, CH)])


_BK = 2000


def _proj_body(x_ref, w_ref, b_ref, o_ref):
    y = jax.nn.relu(jnp.dot(x_ref[...], w_ref[...],
                            preferred_element_type=jnp.float32) + b_ref[...])
    o_ref[0] = y[:, :32]
    o_ref[1] = y[:, 32:]


def _proj_movie(movie_x, proj_W, proj_b):
    return pl.pallas_call(
        _proj_body,
        grid=(NM // _BK,),
        in_specs=[
            pl.BlockSpec((_BK, DF), lambda i: (i, 0)),
            pl.BlockSpec((DF, H), lambda i: (0, 0)),
            pl.BlockSpec((1, H), lambda i: (0, 0)),
        ],
        out_specs=pl.BlockSpec((2, _BK, 32), lambda i: (0, i, 0)),
        out_shape=jax.ShapeDtypeStruct((2, NM, 32), jnp.float32),
    )(movie_x, proj_W, proj_b.reshape(1, H))


def _split_body(x_ref, o_ref):
    o_ref[0] = x_ref[:, :32]
    o_ref[1] = x_ref[:, 32:]


def _split_user(x):
    return pl.pallas_call(
        _split_body,
        grid=(NU // _BK,),
        in_specs=[pl.BlockSpec((_BK, H), lambda i: (i, 0))],
        out_specs=pl.BlockSpec((2, _BK, 32), lambda i: (0, i, 0)),
        out_shape=jax.ShapeDtypeStruct((2, NU, 32), jnp.float32),
    )(x)


def _conv1_body(agg_ref, cnt_ref, x_ref, wl_ref, bl_ref, wr_ref, o_ref):
    inv = 1.0 / jnp.maximum(cnt_ref[...], 1.0)
    wl = wl_ref[...]
    wr = wr_ref[...]
    y = (jnp.dot(agg_ref[0] * inv, wl[:32], preferred_element_type=jnp.float32)
         + jnp.dot(agg_ref[1] * inv, wl[32:],
                   preferred_element_type=jnp.float32)
         + bl_ref[...]
         + jnp.dot(x_ref[0], wr[:32], preferred_element_type=jnp.float32)
         + jnp.dot(x_ref[1], wr[32:], preferred_element_type=jnp.float32))
    y = jax.nn.relu(y)
    o_ref[0] = x_ref[0] + y[:, :32]
    o_ref[1] = x_ref[1] + y[:, 32:]


def _conv1(agg, cnt, x, Wl, bl, Wr, n):
    return pl.pallas_call(
        _conv1_body,
        grid=(n // _BK,),
        in_specs=[
            pl.BlockSpec((2, _BK, 32), lambda i: (0, i, 0)),
            pl.BlockSpec((_BK, 1), lambda i: (i, 0)),
            pl.BlockSpec((2, _BK, 32), lambda i: (0, i, 0)),
            pl.BlockSpec((H, H), lambda i: (0, 0)),
            pl.BlockSpec((1, H), lambda i: (0, 0)),
            pl.BlockSpec((H, H), lambda i: (0, 0)),
        ],
        out_specs=pl.BlockSpec((2, _BK, 32), lambda i: (0, i, 0)),
        out_shape=jax.ShapeDtypeStruct((2, n, 32), jnp.float32),
    )(agg, cnt, x, Wl, bl.reshape(1, H), Wr)


def _conv2_body(agg_ref, cnt_ref, x_ref, wl_ref, bl_ref, wr_ref, w1_ref,
                o_ref):
    inv = 1.0 / jnp.maximum(cnt_ref[...], 1.0)
    wl = wl_ref[...]
    wr = wr_ref[...]
    y = (jnp.dot(agg_ref[0] * inv, wl[:32], preferred_element_type=jnp.float32)
         + jnp.dot(agg_ref[1] * inv, wl[32:],
                   preferred_element_type=jnp.float32)
         + bl_ref[...]
         + jnp.dot(x_ref[0], wr[:32], preferred_element_type=jnp.float32)
         + jnp.dot(x_ref[1], wr[32:], preferred_element_type=jnp.float32))
    nrm = jnp.sqrt(jnp.sum(y * y, axis=-1, keepdims=True))
    yn = y / jnp.maximum(nrm, 1e-12)
    p = jnp.dot(yn, w1_ref[...], preferred_element_type=jnp.float32)
    o_ref[0] = p[:, :32]
    o_ref[1] = p[:, 32:]


def _conv2(agg, cnt, x, Wl, bl, Wr, W1half, n):
    return pl.pallas_call(
        _conv2_body,
        grid=(n // _BK,),
        in_specs=[
            pl.BlockSpec((2, _BK, 32), lambda i: (0, i, 0)),
            pl.BlockSpec((_BK, 1), lambda i: (i, 0)),
            pl.BlockSpec((2, _BK, 32), lambda i: (0, i, 0)),
            pl.BlockSpec((H, H), lambda i: (0, 0)),
            pl.BlockSpec((1, H), lambda i: (0, 0)),
            pl.BlockSpec((H, H), lambda i: (0, 0)),
            pl.BlockSpec((H, H), lambda i: (0, 0)),
        ],
        out_specs=pl.BlockSpec((2, _BK, 32), lambda i: (0, i, 0)),
        out_shape=jax.ShapeDtypeStruct((2, n, 32), jnp.float32),
    )(agg, cnt, x, Wl, bl.reshape(1, H), Wr, W1half)


_ZROWS = LPAD * 32 // 128
_DBK = 2000


def _dec_body(z_ref, b0_ref, b1_ref, m0_ref, m1_ref, b2_ref, o_ref):
    h0 = jax.nn.relu(z_ref[0] + b0_ref[...])
    h1 = jax.nn.relu(z_ref[1] + b1_ref[...])
    o_ref[...] = (jnp.dot(h0, m0_ref[...], preferred_element_type=jnp.float32)
                  + jnp.dot(h1, m1_ref[...],
                            preferred_element_type=jnp.float32)
                  + b2_ref[0, 0])


def _decoder(z, dec_b1, dec_W2, dec_b2):
    b0 = jnp.tile(dec_b1[:32], 4).reshape(1, 128)
    b1 = jnp.tile(dec_b1[32:], 4).reshape(1, 128)
    m0 = jnp.kron(jnp.eye(4, dtype=jnp.float32), dec_W2[:32])
    m1 = jnp.kron(jnp.eye(4, dtype=jnp.float32), dec_W2[32:])
    nrow = EL * 32 // 128
    out = pl.pallas_call(
        _dec_body,
        grid=(nrow // _DBK,),
        in_specs=[
            pl.BlockSpec((2, _DBK, 128), lambda i: (0, i, 0)),
            pl.BlockSpec((1, 128), lambda i: (0, 0)),
            pl.BlockSpec((1, 128), lambda i: (0, 0)),
            pl.BlockSpec((128, 4), lambda i: (0, 0)),
            pl.BlockSpec((128, 4), lambda i: (0, 0)),
            pl.BlockSpec((1, 1), lambda i: (0, 0)),
        ],
        out_specs=pl.BlockSpec((_DBK, 4), lambda i: (i, 0)),
        out_shape=jax.ShapeDtypeStruct((nrow, 4), jnp.float32),
    )(z.reshape(2, _ZROWS, 128), b0, b1, m0, m1, dec_b2.reshape(1, 1))
    return out.reshape(EL, 1)


def _prep_gidx(idx, pad_len):
    p = jnp.zeros((pad_len - idx.shape[0],), jnp.int32)
    a = jnp.concatenate([idx, p])
    return jnp.stack([a, a + 50000])


def _prep_sidx(idx):
    a = jnp.concatenate([idx, jnp.full((EPAD - E,), -1, jnp.int32)])
    s0 = jnp.where((a >= 0) & (a < NPH), a, JUNK2)
    s1 = jnp.where(a >= NPH, a - NPH, JUNK2)
    return jnp.stack([s0, s1]).reshape(2, NCH_E * 8, 128)


def _prep_hidx(idx):
    p = jnp.full((EPAD - E,), JUNK, jnp.int32)
    return jnp.concatenate([idx, p]).reshape(NCH_E * 8, 128)


def kernel(movie_x, edge_index, edge_label_index, user_emb_w, proj_W, proj_b,
           c1_um_Wl, c1_um_bl, c1_um_Wr, c1_mu_Wl, c1_mu_bl, c1_mu_Wr,
           c2_um_Wl, c2_um_bl, c2_um_Wr, c2_mu_Wl, c2_mu_bl, c2_mu_Wr,
           dec_W1, dec_b1, dec_W2, dec_b2):
    src = edge_index[0]
    dst = edge_index[1]
    g_src = _prep_gidx(src, EPAD)
    g_dst = _prep_gidx(dst, EPAD)
    s_dst = _prep_sidx(dst)
    s_src = _prep_sidx(src)
    g_e0 = _prep_gidx(edge_label_index[0], LPAD)
    g_e1 = _prep_gidx(edge_label_index[1], LPAD)

    hist = _sc_hist(jnp.stack([_prep_hidx(dst), _prep_hidx(src)]))
    cnt_m = hist[0, :NM, :1]
    cnt_u = hist[1, :NU, :1]

    xm = _proj_movie(movie_x, proj_W, proj_b)
    xu = _split_user(user_emb_w)

    agg_m1 = _sc_segsum(xu.reshape(2 * NU, 32), g_src, s_dst)
    agg_u1 = _sc_segsum(xm.reshape(2 * NM, 32), g_dst, s_src)

    rm = _conv1(agg_m1, cnt_m, xm, c1_um_Wl, c1_um_bl, c1_um_Wr, NM)
    ru = _conv1(agg_u1, cnt_u, xu, c1_mu_Wl, c1_mu_bl, c1_mu_Wr, NU)

    agg_m2 = _sc_segsum(ru.reshape(2 * NU, 32), g_src, s_dst)
    agg_u2 = _sc_segsum(rm.reshape(2 * NM, 32), g_dst, s_src)

    pm = _conv2(agg_m2, cnt_m, rm, c2_um_Wl, c2_um_bl, c2_um_Wr,
                dec_W1[H:], NM)
    pu = _conv2(agg_u2, cnt_u, ru, c2_mu_Wl, c2_mu_bl, c2_mu_Wr,
                dec_W1[:H], NU)

    z = _sc_decgather(pu.reshape(2 * NU, 32), pm.reshape(2 * NM, 32),
                      g_e0, g_e1)
    return _decoder(z, dec_b1, dec_W2, dec_b2)


# speedup vs baseline: 6.7733x; 1.7664x over previous
import functools

import jax
import jax.numpy as jnp
from jax import lax
from jax.experimental import pallas as pl
from jax.experimental.pallas import tpu as pltpu
from jax.experimental.pallas import tpu_sc as plsc

NU = 50000
NM = 50000
E = 800000
EL = 200000
DF = 128
H = 64

NS = 16
CH = 1024
NCH_E = (E + CH - 1) // CH
EPAD = NCH_E * CH
NT_E = (NCH_E + NS - 1) // NS
NCH_L = (EL + CH - 1) // CH
LPAD = NCH_L * CH
NT_L = (NCH_L + NS - 1) // NS
NP = 51200
RPS = NP // NS
JUNK = 50000

_mesh = plsc.VectorSubcoreMesh(core_axis_name="c", subcore_axis_name="s")
_sc_params = pltpu.CompilerParams(use_tc_tiling_on_sc=False)


def _zero_q16(ref, rows):
    z = jnp.zeros((16,), jnp.float32)

    @pl.loop(0, rows)
    def _(i):
        ref[i, pl.ds(0, 16)] = z


@functools.partial(
    pl.kernel,
    mesh=_mesh,
    compiler_params=_sc_params,
    out_type=jax.ShapeDtypeStruct((2, 2, NP, 16), jnp.float32),
    scratch_types=[
        pltpu.VMEM((CH,), jnp.int32),
        pltpu.VMEM((8, 128), jnp.int32),
        pltpu.VMEM((CH, 16), jnp.float32),
        pltpu.VMEM((RPS, 16), jnp.float32),
        pltpu.VMEM_SHARED((NP, 16), jnp.float32),
        pltpu.SemaphoreType.DMA,
    ],
)
def _sc_segsum(tab, gidx, sidx, out, gv, dv, rows, zbuf, acc, semg):
    c = lax.axis_index("c")
    s = lax.axis_index("s")

    _zero_q16(zbuf, RPS)
    for p in range(2):
        pltpu.sync_copy(zbuf, acc.at[pl.ds(s * RPS, RPS)])
        plsc.subcore_barrier()

        @pl.loop(0, NT_E)
        def _(t):
            j = t * NS + s

            @pl.when(j < NCH_E)
            def _():
                pltpu.sync_copy(gidx.at[c, p, pl.ds(j * CH, CH)], gv)
                pltpu.async_copy(tab.at[gv], rows, semg).wait()
                pltpu.sync_copy(sidx.at[pl.ds(j * 8, 8)], dv)
                for b in range(8):
                    pltpu.sync_copy(rows.at[pl.ds(b * 128, 128)],
                                    acc.at[dv.at[b]], add=True)

        plsc.subcore_barrier()
        pltpu.sync_copy(acc.at[pl.ds(s * RPS, RPS)],
                        out.at[c, p, pl.ds(s * RPS, RPS)])
        plsc.subcore_barrier()


@functools.partial(
    pl.kernel,
    mesh=_mesh,
    compiler_params=_sc_params,
    out_type=jax.ShapeDtypeStruct((2, NP, 16), jnp.float32),
    scratch_types=[
        pltpu.VMEM((8, 128), jnp.int32),
        pltpu.VMEM((128, 16), jnp.float32),
        pltpu.VMEM((RPS, 16), jnp.float32),
        pltpu.VMEM_SHARED((NP, 16), jnp.float32),
    ],
)
def _sc_hist(sidx, out, dv, ones, zbuf, acc):
    c = lax.axis_index("c")
    s = lax.axis_index("s")

    one = jnp.ones((16,), jnp.float32)
    zero = jnp.zeros((16,), jnp.float32)

    @pl.loop(0, 128)
    def _(i):
        ones[i, pl.ds(0, 16)] = one

    @pl.loop(0, RPS)
    def _(i):
        zbuf[i, pl.ds(0, 16)] = zero

    pltpu.sync_copy(zbuf, acc.at[pl.ds(s * RPS, RPS)])
    plsc.subcore_barrier()

    @pl.loop(0, NT_E)
    def _(t):
        j = t * NS + s

        @pl.when(j < NCH_E)
        def _():
            pltpu.sync_copy(sidx.at[c, pl.ds(j * 8, 8)], dv)
            for b in range(8):
                pltpu.sync_copy(ones, acc.at[dv.at[b]], add=True)

    plsc.subcore_barrier()
    pltpu.sync_copy(acc.at[pl.ds(s * RPS, RPS)],
                    out.at[c, pl.ds(s * RPS, RPS)])


@functools.partial(
    pl.kernel,
    mesh=_mesh,
    compiler_params=_sc_params,
    out_type=jax.ShapeDtypeStruct((2, LPAD, 32), jnp.float32),
    scratch_types=[
        pltpu.VMEM((CH,), jnp.int32),
        pltpu.VMEM((CH,), jnp.int32),
        pltpu.VMEM((CH, 32), jnp.float32),
        pltpu.VMEM((CH, 32), jnp.float32),
        pltpu.SemaphoreType.DMA,
        pltpu.SemaphoreType.DMA,
    ],
)
def _sc_decgather(pu, pm, i0, i1, out, iv0, iv1, bu, bm, sem0, sem1):
    c = lax.axis_index("c")
    s = lax.axis_index("s")

    @pl.loop(0, NT_L)
    def _(t):
        j = t * NS + s

        @pl.when(j < NCH_L)
        def _():
            pltpu.sync_copy(i0.at[c, pl.ds(j * CH, CH)], iv0)
            cp0 = pltpu.async_copy(pu.at[iv0], bu, sem0)
            pltpu.sync_copy(i1.at[c, pl.ds(j * CH, CH)], iv1)
            cp1 = pltpu.async_copy(pm.at[iv1], bm, sem1)
            cp0.wait()
            cp1.wait()

            @pl.loop(0, CH)
            def _(r):
                bu[r, pl.ds(0, 16)] = bu[r, pl.ds(0, 16)] + bm[r, pl.ds(0, 16)]
                bu[r, pl.ds(16, 16)] = (bu[r, pl.ds(16, 16)]
                                        + bm[r, pl.ds(16, 16)])

            pltpu.sync_copy(bu, out.at[c, pl.ds(j * CH, CH)])


_BK = 2000


def _proj_body(x_ref, w_ref, b_ref, o_ref):
    y = jax.nn.relu(jnp.dot(x_ref[...], w_ref[...],
                            preferred_element_type=jnp.float32) + b_ref[...])
    for q in range(4):
        o_ref[q] = y[:, 16 * q:16 * (q + 1)]


def _proj_movie(movie_x, proj_W, proj_b):
    return pl.pallas_call(
        _proj_body,
        grid=(NM // _BK,),
        in_specs=[
            pl.BlockSpec((_BK, DF), lambda i: (i, 0)),
            pl.BlockSpec((DF, H), lambda i: (0, 0)),
            pl.BlockSpec((1, H), lambda i: (0, 0)),
        ],
        out_specs=pl.BlockSpec((4, _BK, 16), lambda i: (0, i, 0)),
        out_shape=jax.ShapeDtypeStruct((4, NM, 16), jnp.float32),
    )(movie_x, proj_W, proj_b.reshape(1, H))


def _split_body(x_ref, o_ref):
    for q in range(4):
        o_ref[q] = x_ref[:, 16 * q:16 * (q + 1)]


def _split_user(x):
    return pl.pallas_call(
        _split_body,
        grid=(NU // _BK,),
        in_specs=[pl.BlockSpec((_BK, H), lambda i: (i, 0))],
        out_specs=pl.BlockSpec((4, _BK, 16), lambda i: (0, i, 0)),
        out_shape=jax.ShapeDtypeStruct((4, NU, 16), jnp.float32),
    )(x)


def _sage_linear(agg_ref, cnt_ref, x_ref, wl, bl, wr):
    inv = 1.0 / jnp.maximum(cnt_ref[...], 1.0)
    y = bl
    for q in range(4):
        y = y + jnp.dot(agg_ref[q // 2, q % 2] * inv, wl[16 * q:16 * (q + 1)],
                        preferred_element_type=jnp.float32)
        y = y + jnp.dot(x_ref[q], wr[16 * q:16 * (q + 1)],
                        preferred_element_type=jnp.float32)
    return y


def _conv1_body(agg_ref, cnt_ref, x_ref, wl_ref, bl_ref, wr_ref, o_ref):
    y = jax.nn.relu(_sage_linear(agg_ref, cnt_ref, x_ref,
                                 wl_ref[...], bl_ref[...], wr_ref[...]))
    for q in range(4):
        o_ref[q] = x_ref[q] + y[:, 16 * q:16 * (q + 1)]


def _conv1(agg, cnt, x, Wl, bl, Wr, n):
    return pl.pallas_call(
        _conv1_body,
        grid=(n // _BK,),
        in_specs=[
            pl.BlockSpec((2, 2, _BK, 16), lambda i: (0, 0, i, 0)),
            pl.BlockSpec((_BK, 1), lambda i: (i, 0)),
            pl.BlockSpec((4, _BK, 16), lambda i: (0, i, 0)),
            pl.BlockSpec((H, H), lambda i: (0, 0)),
            pl.BlockSpec((1, H), lambda i: (0, 0)),
            pl.BlockSpec((H, H), lambda i: (0, 0)),
        ],
        out_specs=pl.BlockSpec((4, _BK, 16), lambda i: (0, i, 0)),
        out_shape=jax.ShapeDtypeStruct((4, n, 16), jnp.float32),
    )(agg, cnt, x, Wl, bl.reshape(1, H), Wr)


def _conv2_body(agg_ref, cnt_ref, x_ref, wl_ref, bl_ref, wr_ref, w1_ref,
                o_ref):
    y = _sage_linear(agg_ref, cnt_ref, x_ref,
                     wl_ref[...], bl_ref[...], wr_ref[...])
    nrm = jnp.sqrt(jnp.sum(y * y, axis=-1, keepdims=True))
    yn = y / jnp.maximum(nrm, 1e-12)
    p = jnp.dot(yn, w1_ref[...], preferred_element_type=jnp.float32)
    o_ref[0] = p[:, :32]
    o_ref[1] = p[:, 32:]


def _conv2(agg, cnt, x, Wl, bl, Wr, W1half, n):
    return pl.pallas_call(
        _conv2_body,
        grid=(n // _BK,),
        in_specs=[
            pl.BlockSpec((2, 2, _BK, 16), lambda i: (0, 0, i, 0)),
            pl.BlockSpec((_BK, 1), lambda i: (i, 0)),
            pl.BlockSpec((4, _BK, 16), lambda i: (0, i, 0)),
            pl.BlockSpec((H, H), lambda i: (0, 0)),
            pl.BlockSpec((1, H), lambda i: (0, 0)),
            pl.BlockSpec((H, H), lambda i: (0, 0)),
            pl.BlockSpec((H, H), lambda i: (0, 0)),
        ],
        out_specs=pl.BlockSpec((2, _BK, 32), lambda i: (0, i, 0)),
        out_shape=jax.ShapeDtypeStruct((2, n, 32), jnp.float32),
    )(agg, cnt, x, Wl, bl.reshape(1, H), Wr, W1half)


_ZROWS = LPAD * 32 // 128
_DBK = 2000


def _dec_body(z_ref, b0_ref, b1_ref, m0_ref, m1_ref, b2_ref, o_ref):
    h0 = jax.nn.relu(z_ref[0] + b0_ref[...])
    h1 = jax.nn.relu(z_ref[1] + b1_ref[...])
    o_ref[...] = (jnp.dot(h0, m0_ref[...], preferred_element_type=jnp.float32)
                  + jnp.dot(h1, m1_ref[...],
                            preferred_element_type=jnp.float32)
                  + b2_ref[0, 0])


def _decoder(z, dec_b1, dec_W2, dec_b2):
    b0 = jnp.tile(dec_b1[:32], 4).reshape(1, 128)
    b1 = jnp.tile(dec_b1[32:], 4).reshape(1, 128)
    m0 = jnp.kron(jnp.eye(4, dtype=jnp.float32), dec_W2[:32])
    m1 = jnp.kron(jnp.eye(4, dtype=jnp.float32), dec_W2[32:])
    nrow = EL * 32 // 128
    out = pl.pallas_call(
        _dec_body,
        grid=(nrow // _DBK,),
        in_specs=[
            pl.BlockSpec((2, _DBK, 128), lambda i: (0, i, 0)),
            pl.BlockSpec((1, 128), lambda i: (0, 0)),
            pl.BlockSpec((1, 128), lambda i: (0, 0)),
            pl.BlockSpec((128, 4), lambda i: (0, 0)),
            pl.BlockSpec((128, 4), lambda i: (0, 0)),
            pl.BlockSpec((1, 1), lambda i: (0, 0)),
        ],
        out_specs=pl.BlockSpec((_DBK, 4), lambda i: (i, 0)),
        out_shape=jax.ShapeDtypeStruct((nrow, 4), jnp.float32),
    )(z.reshape(2, _ZROWS, 128), b0, b1, m0, m1, dec_b2.reshape(1, 1))
    return out.reshape(EL, 1)


def _prep_gidx4(idx):
    p = jnp.zeros((EPAD - idx.shape[0],), jnp.int32)
    a = jnp.concatenate([idx, p])
    return (a[None, None, :]
            + jnp.arange(4, dtype=jnp.int32).reshape(2, 2, 1) * 50000)


def _prep_gidx2(idx):
    p = jnp.zeros((LPAD - idx.shape[0],), jnp.int32)
    a = jnp.concatenate([idx, p])
    return jnp.stack([a, a + 50000])


def _prep_sidx(idx):
    p = jnp.full((EPAD - E,), JUNK, jnp.int32)
    return jnp.concatenate([idx, p]).reshape(NCH_E * 8, 128)


def kernel(movie_x, edge_index, edge_label_index, user_emb_w, proj_W, proj_b,
           c1_um_Wl, c1_um_bl, c1_um_Wr, c1_mu_Wl, c1_mu_bl, c1_mu_Wr,
           c2_um_Wl, c2_um_bl, c2_um_Wr, c2_mu_Wl, c2_mu_bl, c2_mu_Wr,
           dec_W1, dec_b1, dec_W2, dec_b2):
    src = edge_index[0]
    dst = edge_index[1]
    g_src = _prep_gidx4(src)
    g_dst = _prep_gidx4(dst)
    s_dst = _prep_sidx(dst)
    s_src = _prep_sidx(src)
    g_e0 = _prep_gidx2(edge_label_index[0])
    g_e1 = _prep_gidx2(edge_label_index[1])

    hist = _sc_hist(jnp.stack([s_dst, s_src]))
    cnt_m = hist[0, :NM, :1]
    cnt_u = hist[1, :NU, :1]

    xm = _proj_movie(movie_x, proj_W, proj_b)
    xu = _split_user(user_emb_w)

    agg_m1 = _sc_segsum(xu.reshape(4 * NU, 16), g_src, s_dst)
    agg_u1 = _sc_segsum(xm.reshape(4 * NM, 16), g_dst, s_src)

    rm = _conv1(agg_m1, cnt_m, xm, c1_um_Wl, c1_um_bl, c1_um_Wr, NM)
    ru = _conv1(agg_u1, cnt_u, xu, c1_mu_Wl, c1_mu_bl, c1_mu_Wr, NU)

    agg_m2 = _sc_segsum(ru.reshape(4 * NU, 16), g_src, s_dst)
    agg_u2 = _sc_segsum(rm.reshape(4 * NM, 16), g_dst, s_src)

    pm = _conv2(agg_m2, cnt_m, rm, c2_um_Wl, c2_um_bl, c2_um_Wr,
                dec_W1[H:], NM)
    pu = _conv2(agg_u2, cnt_u, ru, c2_mu_Wl, c2_mu_bl, c2_mu_Wr,
                dec_W1[:H], NU)

    z = _sc_decgather(pu.reshape(2 * NU, 32), pm.reshape(2 * NM, 32),
                      g_e0, g_e1)
    return _decoder(z, dec_b1, dec_W2, dec_b2)


# speedup vs baseline: 6.9696x; 1.0290x over previous
import functools

import jax
import jax.numpy as jnp
from jax import lax
from jax.experimental import pallas as pl
from jax.experimental.pallas import tpu as pltpu
from jax.experimental.pallas import tpu_sc as plsc

NU = 50000
NM = 50000
E = 800000
EL = 200000
DF = 128
H = 64

NS = 16
CH = 1024
NCH_E = (E + CH - 1) // CH
EPAD = NCH_E * CH
NT_E = (NCH_E + NS - 1) // NS
NCH_L = (EL + CH - 1) // CH
LPAD = NCH_L * CH
NT_L = (NCH_L + NS - 1) // NS
NP = 51200
RPS = NP // NS
JUNK = 50000

_mesh = plsc.VectorSubcoreMesh(core_axis_name="c", subcore_axis_name="s")
_sc_params = pltpu.CompilerParams(use_tc_tiling_on_sc=False)


def _zero_q16(ref, rows):
    z = jnp.zeros((16,), jnp.float32)

    @pl.loop(0, rows)
    def _(i):
        ref[i, pl.ds(0, 16)] = z


@functools.partial(
    pl.kernel,
    mesh=_mesh,
    compiler_params=_sc_params,
    out_type=jax.ShapeDtypeStruct((2, 2, NP, 16), jnp.float32),
    scratch_types=[
        pltpu.VMEM((CH,), jnp.int32),
        pltpu.VMEM((CH,), jnp.int32),
        pltpu.VMEM((CH, 16), jnp.float32),
        pltpu.VMEM((RPS, 16), jnp.float32),
        pltpu.VMEM_SHARED((NP, 16), jnp.float32),
        pltpu.SemaphoreType.DMA,
    ],
)
def _sc_segsum(tab, gidx, sidx, out, gv, dv, rows, zbuf, acc, semg):
    c = lax.axis_index("c")
    s = lax.axis_index("s")

    _zero_q16(zbuf, RPS)
    for p in range(2):
        pltpu.sync_copy(zbuf, acc.at[pl.ds(s * RPS, RPS)])
        plsc.subcore_barrier()

        @pl.loop(0, NT_E)
        def _(t):
            j = t * NS + s

            @pl.when(j < NCH_E)
            def _():
                pltpu.sync_copy(gidx.at[c, p, pl.ds(j * CH, CH)], gv)
                pltpu.async_copy(tab.at[gv], rows, semg).wait()
                pltpu.sync_copy(sidx.at[pl.ds(j * CH, CH)], dv)
                pltpu.sync_copy(rows, acc.at[dv], add=True)

        plsc.subcore_barrier()
        pltpu.sync_copy(acc.at[pl.ds(s * RPS, RPS)],
                        out.at[c, p, pl.ds(s * RPS, RPS)])
        plsc.subcore_barrier()


@functools.partial(
    pl.kernel,
    mesh=_mesh,
    compiler_params=_sc_params,
    out_type=jax.ShapeDtypeStruct((2, NP, 16), jnp.float32),
    scratch_types=[
        pltpu.VMEM((CH,), jnp.int32),
        pltpu.VMEM((CH, 16), jnp.float32),
        pltpu.VMEM((RPS, 16), jnp.float32),
        pltpu.VMEM_SHARED((NP, 16), jnp.float32),
    ],
)
def _sc_hist(sidx, out, dv, ones, zbuf, acc):
    c = lax.axis_index("c")
    s = lax.axis_index("s")

    one = jnp.ones((16,), jnp.float32)
    zero = jnp.zeros((16,), jnp.float32)

    @pl.loop(0, CH)
    def _(i):
        ones[i, pl.ds(0, 16)] = one

    @pl.loop(0, RPS)
    def _(i):
        zbuf[i, pl.ds(0, 16)] = zero

    pltpu.sync_copy(zbuf, acc.at[pl.ds(s * RPS, RPS)])
    plsc.subcore_barrier()

    @pl.loop(0, NT_E)
    def _(t):
        j = t * NS + s

        @pl.when(j < NCH_E)
        def _():
            pltpu.sync_copy(sidx.at[c, pl.ds(j * CH, CH)], dv)
            pltpu.sync_copy(ones, acc.at[dv], add=True)

    plsc.subcore_barrier()
    pltpu.sync_copy(acc.at[pl.ds(s * RPS, RPS)],
                    out.at[c, pl.ds(s * RPS, RPS)])


@functools.partial(
    pl.kernel,
    mesh=_mesh,
    compiler_params=_sc_params,
    out_type=jax.ShapeDtypeStruct((2, LPAD, 32), jnp.float32),
    scratch_types=[
        pltpu.VMEM((CH,), jnp.int32),
        pltpu.VMEM((CH,), jnp.int32),
        pltpu.VMEM((CH, 32), jnp.float32),
        pltpu.VMEM((CH, 32), jnp.float32),
        pltpu.SemaphoreType.DMA,
        pltpu.SemaphoreType.DMA,
    ],
)
def _sc_decgather(pu, pm, i0, i1, out, iv0, iv1, bu, bm, sem0, sem1):
    c = lax.axis_index("c")
    s = lax.axis_index("s")

    @pl.loop(0, NT_L)
    def _(t):
        j = t * NS + s

        @pl.when(j < NCH_L)
        def _():
            pltpu.sync_copy(i0.at[c, pl.ds(j * CH, CH)], iv0)
            cp0 = pltpu.async_copy(pu.at[iv0], bu, sem0)
            pltpu.sync_copy(i1.at[c, pl.ds(j * CH, CH)], iv1)
            cp1 = pltpu.async_copy(pm.at[iv1], bm, sem1)
            cp0.wait()
            cp1.wait()

            @pl.loop(0, CH)
            def _(r):
                bu[r, pl.ds(0, 16)] = bu[r, pl.ds(0, 16)] + bm[r, pl.ds(0, 16)]
                bu[r, pl.ds(16, 16)] = (bu[r, pl.ds(16, 16)]
                                        + bm[r, pl.ds(16, 16)])

            pltpu.sync_copy(bu, out.at[c, pl.ds(j * CH, CH)])


_BK = 2000


def _proj_body(x_ref, w_ref, b_ref, o_ref):
    y = jax.nn.relu(jnp.dot(x_ref[...], w_ref[...],
                            preferred_element_type=jnp.float32) + b_ref[...])
    for q in range(4):
        o_ref[q] = y[:, 16 * q:16 * (q + 1)]


def _proj_movie(movie_x, proj_W, proj_b):
    return pl.pallas_call(
        _proj_body,
        grid=(NM // _BK,),
        in_specs=[
            pl.BlockSpec((_BK, DF), lambda i: (i, 0)),
            pl.BlockSpec((DF, H), lambda i: (0, 0)),
            pl.BlockSpec((1, H), lambda i: (0, 0)),
        ],
        out_specs=pl.BlockSpec((4, _BK, 16), lambda i: (0, i, 0)),
        out_shape=jax.ShapeDtypeStruct((4, NM, 16), jnp.float32),
    )(movie_x, proj_W, proj_b.reshape(1, H))


def _split_body(x_ref, o_ref):
    for q in range(4):
        o_ref[q] = x_ref[:, 16 * q:16 * (q + 1)]


def _split_user(x):
    return pl.pallas_call(
        _split_body,
        grid=(NU // _BK,),
        in_specs=[pl.BlockSpec((_BK, H), lambda i: (i, 0))],
        out_specs=pl.BlockSpec((4, _BK, 16), lambda i: (0, i, 0)),
        out_shape=jax.ShapeDtypeStruct((4, NU, 16), jnp.float32),
    )(x)


def _sage_linear(agg_ref, cnt_ref, x_ref, wl, bl, wr):
    inv = 1.0 / jnp.maximum(cnt_ref[...], 1.0)
    y = bl
    for q in range(4):
        y = y + jnp.dot(agg_ref[q // 2, q % 2] * inv, wl[16 * q:16 * (q + 1)],
                        preferred_element_type=jnp.float32)
        y = y + jnp.dot(x_ref[q], wr[16 * q:16 * (q + 1)],
                        preferred_element_type=jnp.float32)
    return y


def _conv1_body(agg_ref, cnt_ref, x_ref, wl_ref, bl_ref, wr_ref, o_ref):
    y = jax.nn.relu(_sage_linear(agg_ref, cnt_ref, x_ref,
                                 wl_ref[...], bl_ref[...], wr_ref[...]))
    for q in range(4):
        o_ref[q] = x_ref[q] + y[:, 16 * q:16 * (q + 1)]


def _conv1(agg, cnt, x, Wl, bl, Wr, n):
    return pl.pallas_call(
        _conv1_body,
        grid=(n // _BK,),
        in_specs=[
            pl.BlockSpec((2, 2, _BK, 16), lambda i: (0, 0, i, 0)),
            pl.BlockSpec((_BK, 1), lambda i: (i, 0)),
            pl.BlockSpec((4, _BK, 16), lambda i: (0, i, 0)),
            pl.BlockSpec((H, H), lambda i: (0, 0)),
            pl.BlockSpec((1, H), lambda i: (0, 0)),
            pl.BlockSpec((H, H), lambda i: (0, 0)),
        ],
        out_specs=pl.BlockSpec((4, _BK, 16), lambda i: (0, i, 0)),
        out_shape=jax.ShapeDtypeStruct((4, n, 16), jnp.float32),
    )(agg, cnt, x, Wl, bl.reshape(1, H), Wr)


def _conv2_body(agg_ref, cnt_ref, x_ref, wl_ref, bl_ref, wr_ref, w1_ref,
                o_ref):
    y = _sage_linear(agg_ref, cnt_ref, x_ref,
                     wl_ref[...], bl_ref[...], wr_ref[...])
    nrm = jnp.sqrt(jnp.sum(y * y, axis=-1, keepdims=True))
    yn = y / jnp.maximum(nrm, 1e-12)
    p = jnp.dot(yn, w1_ref[...], preferred_element_type=jnp.float32)
    o_ref[0] = p[:, :32]
    o_ref[1] = p[:, 32:]


def _conv2(agg, cnt, x, Wl, bl, Wr, W1half, n):
    return pl.pallas_call(
        _conv2_body,
        grid=(n // _BK,),
        in_specs=[
            pl.BlockSpec((2, 2, _BK, 16), lambda i: (0, 0, i, 0)),
            pl.BlockSpec((_BK, 1), lambda i: (i, 0)),
            pl.BlockSpec((4, _BK, 16), lambda i: (0, i, 0)),
            pl.BlockSpec((H, H), lambda i: (0, 0)),
            pl.BlockSpec((1, H), lambda i: (0, 0)),
            pl.BlockSpec((H, H), lambda i: (0, 0)),
            pl.BlockSpec((H, H), lambda i: (0, 0)),
        ],
        out_specs=pl.BlockSpec((2, _BK, 32), lambda i: (0, i, 0)),
        out_shape=jax.ShapeDtypeStruct((2, n, 32), jnp.float32),
    )(agg, cnt, x, Wl, bl.reshape(1, H), Wr, W1half)


_ZROWS = LPAD * 32 // 128
_DBK = 2000


def _dec_body(z_ref, b0_ref, b1_ref, m0_ref, m1_ref, b2_ref, o_ref):
    h0 = jax.nn.relu(z_ref[0] + b0_ref[...])
    h1 = jax.nn.relu(z_ref[1] + b1_ref[...])
    o_ref[...] = (jnp.dot(h0, m0_ref[...], preferred_element_type=jnp.float32)
                  + jnp.dot(h1, m1_ref[...],
                            preferred_element_type=jnp.float32)
                  + b2_ref[0, 0])


def _decoder(z, dec_b1, dec_W2, dec_b2):
    b0 = jnp.tile(dec_b1[:32], 4).reshape(1, 128)
    b1 = jnp.tile(dec_b1[32:], 4).reshape(1, 128)
    m0 = jnp.kron(jnp.eye(4, dtype=jnp.float32), dec_W2[:32])
    m1 = jnp.kron(jnp.eye(4, dtype=jnp.float32), dec_W2[32:])
    nrow = EL * 32 // 128
    out = pl.pallas_call(
        _dec_body,
        grid=(nrow // _DBK,),
        in_specs=[
            pl.BlockSpec((2, _DBK, 128), lambda i: (0, i, 0)),
            pl.BlockSpec((1, 128), lambda i: (0, 0)),
            pl.BlockSpec((1, 128), lambda i: (0, 0)),
            pl.BlockSpec((128, 4), lambda i: (0, 0)),
            pl.BlockSpec((128, 4), lambda i: (0, 0)),
            pl.BlockSpec((1, 1), lambda i: (0, 0)),
        ],
        out_specs=pl.BlockSpec((_DBK, 4), lambda i: (i, 0)),
        out_shape=jax.ShapeDtypeStruct((nrow, 4), jnp.float32),
    )(z.reshape(2, _ZROWS, 128), b0, b1, m0, m1, dec_b2.reshape(1, 1))
    return out.reshape(EL, 1)


def _prep_gidx4(idx):
    p = jnp.zeros((EPAD - idx.shape[0],), jnp.int32)
    a = jnp.concatenate([idx, p])
    return (a[None, None, :]
            + jnp.arange(4, dtype=jnp.int32).reshape(2, 2, 1) * 50000)


def _prep_gidx2(idx):
    p = jnp.zeros((LPAD - idx.shape[0],), jnp.int32)
    a = jnp.concatenate([idx, p])
    return jnp.stack([a, a + 50000])


def _prep_sidx(idx):
    p = jnp.full((EPAD - E,), JUNK, jnp.int32)
    return jnp.concatenate([idx, p])


def kernel(movie_x, edge_index, edge_label_index, user_emb_w, proj_W, proj_b,
           c1_um_Wl, c1_um_bl, c1_um_Wr, c1_mu_Wl, c1_mu_bl, c1_mu_Wr,
           c2_um_Wl, c2_um_bl, c2_um_Wr, c2_mu_Wl, c2_mu_bl, c2_mu_Wr,
           dec_W1, dec_b1, dec_W2, dec_b2):
    src = edge_index[0]
    dst = edge_index[1]
    g_src = _prep_gidx4(src)
    g_dst = _prep_gidx4(dst)
    s_dst = _prep_sidx(dst)
    s_src = _prep_sidx(src)
    g_e0 = _prep_gidx2(edge_label_index[0])
    g_e1 = _prep_gidx2(edge_label_index[1])

    hist = _sc_hist(jnp.stack([s_dst, s_src]))
    cnt_m = hist[0, :NM, :1]
    cnt_u = hist[1, :NU, :1]

    xm = _proj_movie(movie_x, proj_W, proj_b)
    xu = _split_user(user_emb_w)

    agg_m1 = _sc_segsum(xu.reshape(4 * NU, 16), g_src, s_dst)
    agg_u1 = _sc_segsum(xm.reshape(4 * NM, 16), g_dst, s_src)

    rm = _conv1(agg_m1, cnt_m, xm, c1_um_Wl, c1_um_bl, c1_um_Wr, NM)
    ru = _conv1(agg_u1, cnt_u, xu, c1_mu_Wl, c1_mu_bl, c1_mu_Wr, NU)

    agg_m2 = _sc_segsum(ru.reshape(4 * NU, 16), g_src, s_dst)
    agg_u2 = _sc_segsum(rm.reshape(4 * NM, 16), g_dst, s_src)

    pm = _conv2(agg_m2, cnt_m, rm, c2_um_Wl, c2_um_bl, c2_um_Wr,
                dec_W1[H:], NM)
    pu = _conv2(agg_u2, cnt_u, ru, c2_mu_Wl, c2_mu_bl, c2_mu_Wr,
                dec_W1[:H], NU)

    z = _sc_decgather(pu.reshape(2 * NU, 32), pm.reshape(2 * NM, 32),
                      g_e0, g_e1)
    return _decoder(z, dec_b1, dec_W2, dec_b2)


# speedup vs baseline: 7.4411x; 1.0677x over previous
import functools

import jax
import jax.numpy as jnp
from jax import lax
from jax.experimental import pallas as pl
from jax.experimental.pallas import tpu as pltpu
from jax.experimental.pallas import tpu_sc as plsc

NU = 50000
NM = 50000
E = 800000
EL = 200000
DF = 128
H = 64

NS = 16
CH = 1024
NTC = 49
NCH_E = NS * NTC
EPAD = NCH_E * CH
NCH_L = (EL + CH - 1) // CH
LPAD = NCH_L * CH
NT_L = (NCH_L + NS - 1) // NS
NP = 51200
RPS = NP // NS
JUNK = 50000

_mesh = plsc.VectorSubcoreMesh(core_axis_name="c", subcore_axis_name="s")
_sc_params = pltpu.CompilerParams(use_tc_tiling_on_sc=False)


def _zero_q16(ref, rows):
    z = jnp.zeros((16,), jnp.float32)

    @pl.loop(0, rows)
    def _(i):
        ref[i, pl.ds(0, 16)] = z


def _zero_acc_via(buf, acc, s):
    _zero_q16(buf, CH)
    for k in range(3):
        pltpu.sync_copy(buf, acc.at[pl.ds(s * RPS + k * CH, CH)])
    pltpu.sync_copy(buf.at[pl.ds(0, RPS - 3 * CH)],
                    acc.at[pl.ds(s * RPS + 3 * CH, RPS - 3 * CH)])


@functools.partial(
    pl.kernel,
    mesh=_mesh,
    compiler_params=_sc_params,
    out_type=jax.ShapeDtypeStruct((2, 2, NP, 16), jnp.float32),
    scratch_types=[
        pltpu.VMEM((NTC, CH), jnp.int32),
        pltpu.VMEM((CH,), jnp.int32),
        pltpu.VMEM((CH, 16), jnp.float32),
        pltpu.VMEM_SHARED((NP, 16), jnp.float32),
        pltpu.SemaphoreType.DMA,
    ],
)
def _sc_segsum(tab, gidx, sidx, out, gsta, dv, rows, acc, semg):
    c = lax.axis_index("c")
    s = lax.axis_index("s")

    for p in range(2):
        _zero_acc_via(rows, acc, s)
        pltpu.sync_copy(gidx.at[c, p, pl.ds(s * NTC, NTC)], gsta)
        plsc.subcore_barrier()

        @pl.loop(0, NTC)
        def _(t):
            pltpu.async_copy(tab.at[gsta.at[t]], rows, semg).wait()
            pltpu.sync_copy(sidx.at[s * NTC + t], dv)
            pltpu.sync_copy(rows, acc.at[dv], add=True)

        plsc.subcore_barrier()
        pltpu.sync_copy(acc.at[pl.ds(s * RPS, RPS)],
                        out.at[c, p, pl.ds(s * RPS, RPS)])
        plsc.subcore_barrier()


@functools.partial(
    pl.kernel,
    mesh=_mesh,
    compiler_params=_sc_params,
    out_type=jax.ShapeDtypeStruct((2, NP, 16), jnp.float32),
    scratch_types=[
        pltpu.VMEM((CH,), jnp.int32),
        pltpu.VMEM((CH, 16), jnp.float32),
        pltpu.VMEM_SHARED((NP, 16), jnp.float32),
    ],
)
def _sc_hist(sidx, out, dv, ones, acc):
    c = lax.axis_index("c")
    s = lax.axis_index("s")

    _zero_acc_via(ones, acc, s)

    one = jnp.ones((16,), jnp.float32)

    @pl.loop(0, CH)
    def _(i):
        ones[i, pl.ds(0, 16)] = one

    plsc.subcore_barrier()

    @pl.loop(0, NTC)
    def _(t):
        pltpu.sync_copy(sidx.at[c, s * NTC + t], dv)
        pltpu.sync_copy(ones, acc.at[dv], add=True)

    plsc.subcore_barrier()
    pltpu.sync_copy(acc.at[pl.ds(s * RPS, RPS)],
                    out.at[c, pl.ds(s * RPS, RPS)])


@functools.partial(
    pl.kernel,
    mesh=_mesh,
    compiler_params=_sc_params,
    out_type=jax.ShapeDtypeStruct((2, LPAD, 32), jnp.float32),
    scratch_types=[
        pltpu.VMEM((CH,), jnp.int32),
        pltpu.VMEM((CH,), jnp.int32),
        pltpu.VMEM((CH, 32), jnp.float32),
        pltpu.VMEM((CH, 32), jnp.float32),
        pltpu.SemaphoreType.DMA,
        pltpu.SemaphoreType.DMA,
    ],
)
def _sc_decgather(pu, pm, i0, i1, out, iv0, iv1, bu, bm, sem0, sem1):
    c = lax.axis_index("c")
    s = lax.axis_index("s")

    @pl.loop(0, NT_L)
    def _(t):
        j = t * NS + s

        @pl.when(j < NCH_L)
        def _():
            pltpu.sync_copy(i0.at[c, pl.ds(j * CH, CH)], iv0)
            cp0 = pltpu.async_copy(pu.at[iv0], bu, sem0)
            pltpu.sync_copy(i1.at[c, pl.ds(j * CH, CH)], iv1)
            cp1 = pltpu.async_copy(pm.at[iv1], bm, sem1)
            cp0.wait()
            cp1.wait()

            @pl.loop(0, CH)
            def _(r):
                bu[r, pl.ds(0, 16)] = bu[r, pl.ds(0, 16)] + bm[r, pl.ds(0, 16)]
                bu[r, pl.ds(16, 16)] = (bu[r, pl.ds(16, 16)]
                                        + bm[r, pl.ds(16, 16)])

            pltpu.sync_copy(bu, out.at[c, pl.ds(j * CH, CH)])


_BK = 2000


def _proj_body(x_ref, w_ref, b_ref, o_ref):
    y = jax.nn.relu(jnp.dot(x_ref[...], w_ref[...],
                            preferred_element_type=jnp.float32) + b_ref[...])
    for q in range(4):
        o_ref[q] = y[:, 16 * q:16 * (q + 1)]


def _proj_movie(movie_x, proj_W, proj_b):
    return pl.pallas_call(
        _proj_body,
        grid=(NM // _BK,),
        in_specs=[
            pl.BlockSpec((_BK, DF), lambda i: (i, 0)),
            pl.BlockSpec((DF, H), lambda i: (0, 0)),
            pl.BlockSpec((1, H), lambda i: (0, 0)),
        ],
        out_specs=pl.BlockSpec((4, _BK, 16), lambda i: (0, i, 0)),
        out_shape=jax.ShapeDtypeStruct((4, NM, 16), jnp.float32),
    )(movie_x, proj_W, proj_b.reshape(1, H))


def _split_body(x_ref, o_ref):
    for q in range(4):
        o_ref[q] = x_ref[:, 16 * q:16 * (q + 1)]


def _split_user(x):
    return pl.pallas_call(
        _split_body,
        grid=(NU // _BK,),
        in_specs=[pl.BlockSpec((_BK, H), lambda i: (i, 0))],
        out_specs=pl.BlockSpec((4, _BK, 16), lambda i: (0, i, 0)),
        out_shape=jax.ShapeDtypeStruct((4, NU, 16), jnp.float32),
    )(x)


def _sage_linear(agg_ref, cnt_ref, x_ref, wl, bl, wr):
    inv = 1.0 / jnp.maximum(cnt_ref[...], 1.0)
    y = bl
    for q in range(4):
        y = y + jnp.dot(agg_ref[q // 2, q % 2] * inv, wl[16 * q:16 * (q + 1)],
                        preferred_element_type=jnp.float32)
        y = y + jnp.dot(x_ref[q], wr[16 * q:16 * (q + 1)],
                        preferred_element_type=jnp.float32)
    return y


def _conv1_body(agg_ref, cnt_ref, x_ref, wl_ref, bl_ref, wr_ref, o_ref):
    y = jax.nn.relu(_sage_linear(agg_ref, cnt_ref, x_ref,
                                 wl_ref[...], bl_ref[...], wr_ref[...]))
    for q in range(4):
        o_ref[q] = x_ref[q] + y[:, 16 * q:16 * (q + 1)]


def _conv1(agg, cnt, x, Wl, bl, Wr, n):
    return pl.pallas_call(
        _conv1_body,
        grid=(n // _BK,),
        in_specs=[
            pl.BlockSpec((2, 2, _BK, 16), lambda i: (0, 0, i, 0)),
            pl.BlockSpec((_BK, 1), lambda i: (i, 0)),
            pl.BlockSpec((4, _BK, 16), lambda i: (0, i, 0)),
            pl.BlockSpec((H, H), lambda i: (0, 0)),
            pl.BlockSpec((1, H), lambda i: (0, 0)),
            pl.BlockSpec((H, H), lambda i: (0, 0)),
        ],
        out_specs=pl.BlockSpec((4, _BK, 16), lambda i: (0, i, 0)),
        out_shape=jax.ShapeDtypeStruct((4, n, 16), jnp.float32),
    )(agg, cnt, x, Wl, bl.reshape(1, H), Wr)


def _conv2_body(agg_ref, cnt_ref, x_ref, wl_ref, bl_ref, wr_ref, w1_ref,
                o_ref):
    y = _sage_linear(agg_ref, cnt_ref, x_ref,
                     wl_ref[...], bl_ref[...], wr_ref[...])
    nrm = jnp.sqrt(jnp.sum(y * y, axis=-1, keepdims=True))
    yn = y / jnp.maximum(nrm, 1e-12)
    p = jnp.dot(yn, w1_ref[...], preferred_element_type=jnp.float32)
    o_ref[0] = p[:, :32]
    o_ref[1] = p[:, 32:]


def _conv2(agg, cnt, x, Wl, bl, Wr, W1half, n):
    return pl.pallas_call(
        _conv2_body,
        grid=(n // _BK,),
        in_specs=[
            pl.BlockSpec((2, 2, _BK, 16), lambda i: (0, 0, i, 0)),
            pl.BlockSpec((_BK, 1), lambda i: (i, 0)),
            pl.BlockSpec((4, _BK, 16), lambda i: (0, i, 0)),
            pl.BlockSpec((H, H), lambda i: (0, 0)),
            pl.BlockSpec((1, H), lambda i: (0, 0)),
            pl.BlockSpec((H, H), lambda i: (0, 0)),
            pl.BlockSpec((H, H), lambda i: (0, 0)),
        ],
        out_specs=pl.BlockSpec((2, _BK, 32), lambda i: (0, i, 0)),
        out_shape=jax.ShapeDtypeStruct((2, n, 32), jnp.float32),
    )(agg, cnt, x, Wl, bl.reshape(1, H), Wr, W1half)


_ZROWS = LPAD * 32 // 128
_DBK = 2000


def _dec_body(z_ref, b0_ref, b1_ref, m0_ref, m1_ref, b2_ref, o_ref):
    h0 = jax.nn.relu(z_ref[0] + b0_ref[...])
    h1 = jax.nn.relu(z_ref[1] + b1_ref[...])
    o_ref[...] = (jnp.dot(h0, m0_ref[...], preferred_element_type=jnp.float32)
                  + jnp.dot(h1, m1_ref[...],
                            preferred_element_type=jnp.float32)
                  + b2_ref[0, 0])


def _decoder(z, dec_b1, dec_W2, dec_b2):
    b0 = jnp.tile(dec_b1[:32], 4).reshape(1, 128)
    b1 = jnp.tile(dec_b1[32:], 4).reshape(1, 128)
    m0 = jnp.kron(jnp.eye(4, dtype=jnp.float32), dec_W2[:32])
    m1 = jnp.kron(jnp.eye(4, dtype=jnp.float32), dec_W2[32:])
    nrow = EL * 32 // 128
    out = pl.pallas_call(
        _dec_body,
        grid=(nrow // _DBK,),
        in_specs=[
            pl.BlockSpec((2, _DBK, 128), lambda i: (0, i, 0)),
            pl.BlockSpec((1, 128), lambda i: (0, 0)),
            pl.BlockSpec((1, 128), lambda i: (0, 0)),
            pl.BlockSpec((128, 4), lambda i: (0, 0)),
            pl.BlockSpec((128, 4), lambda i: (0, 0)),
            pl.BlockSpec((1, 1), lambda i: (0, 0)),
        ],
        out_specs=pl.BlockSpec((_DBK, 4), lambda i: (i, 0)),
        out_shape=jax.ShapeDtypeStruct((nrow, 4), jnp.float32),
    )(z.reshape(2, _ZROWS, 128), b0, b1, m0, m1, dec_b2.reshape(1, 1))
    return out.reshape(EL, 1)


def _prep_gidx4(idx):
    p = jnp.zeros((EPAD - idx.shape[0],), jnp.int32)
    a = jnp.concatenate([idx, p])
    return (a[None, None, :]
            + jnp.arange(4, dtype=jnp.int32).reshape(2, 2, 1) * 50000
            ).reshape(2, 2, NCH_E, CH)


def _prep_gidx2(idx):
    p = jnp.zeros((LPAD - idx.shape[0],), jnp.int32)
    a = jnp.concatenate([idx, p])
    return jnp.stack([a, a + 50000])


def _prep_sidx(idx):
    p = jnp.full((EPAD - E,), JUNK, jnp.int32)
    return jnp.concatenate([idx, p]).reshape(NCH_E, CH)


def kernel(movie_x, edge_index, edge_label_index, user_emb_w, proj_W, proj_b,
           c1_um_Wl, c1_um_bl, c1_um_Wr, c1_mu_Wl, c1_mu_bl, c1_mu_Wr,
           c2_um_Wl, c2_um_bl, c2_um_Wr, c2_mu_Wl, c2_mu_bl, c2_mu_Wr,
           dec_W1, dec_b1, dec_W2, dec_b2):
    src = edge_index[0]
    dst = edge_index[1]
    g_src = _prep_gidx4(src)
    g_dst = _prep_gidx4(dst)
    s_dst = _prep_sidx(dst)
    s_src = _prep_sidx(src)
    g_e0 = _prep_gidx2(edge_label_index[0])
    g_e1 = _prep_gidx2(edge_label_index[1])

    hist = _sc_hist(jnp.stack([s_dst, s_src]))
    cnt_m = hist[0, :NM, :1]
    cnt_u = hist[1, :NU, :1]

    xm = _proj_movie(movie_x, proj_W, proj_b)
    xu = _split_user(user_emb_w)

    agg_m1 = _sc_segsum(xu.reshape(4 * NU, 16), g_src, s_dst)
    agg_u1 = _sc_segsum(xm.reshape(4 * NM, 16), g_dst, s_src)

    rm = _conv1(agg_m1, cnt_m, xm, c1_um_Wl, c1_um_bl, c1_um_Wr, NM)
    ru = _conv1(agg_u1, cnt_u, xu, c1_mu_Wl, c1_mu_bl, c1_mu_Wr, NU)

    agg_m2 = _sc_segsum(ru.reshape(4 * NU, 16), g_src, s_dst)
    agg_u2 = _sc_segsum(rm.reshape(4 * NM, 16), g_dst, s_src)

    pm = _conv2(agg_m2, cnt_m, rm, c2_um_Wl, c2_um_bl, c2_um_Wr,
                dec_W1[H:], NM)
    pu = _conv2(agg_u2, cnt_u, ru, c2_mu_Wl, c2_mu_bl, c2_mu_Wr,
                dec_W1[:H], NU)

    z = _sc_decgather(pu.reshape(2 * NU, 32), pm.reshape(2 * NM, 32),
                      g_e0, g_e1)
    return _decoder(z, dec_b1, dec_W2, dec_b2)
